# baseline (speedup 1.0000x reference)
import functools

import jax
import jax.numpy as jnp
from jax import lax
from jax.experimental import pallas as pl
from jax.experimental.pallas import tpu as pltpu
from jax.experimental.pallas import tpu_sc as plsc

B = 16384
D = 64
NC = 2
NS = 16
L = 16
NW = NC * NS
B_W = B // NW
NCHUNK = 4
CB = B_W // NCHUNK


def _kge_body(s_hbm, p_hbm, o_hbm, ent_hbm, rel_hbm, out_hbm,
              s_idx, p_idx, o_idx, s_rows, p_rows, o_rows, out_v, tmat, sem):
    wid = lax.axis_index("s") * NC + lax.axis_index("c")
    base = wid * NCHUNK

    pltpu.sync_copy(s_hbm.at[pl.ds(base, NCHUNK)], s_idx)
    pltpu.sync_copy(p_hbm.at[pl.ds(base, NCHUNK)], p_idx)
    pltpu.sync_copy(o_hbm.at[pl.ds(base, NCHUNK)], o_idx)

    copies = []
    for c in range(NCHUNK):
        dst = pl.ds(c * CB, CB)
        copies.append(pltpu.async_copy(ent_hbm.at[s_idx.at[c]], s_rows.at[dst], sem))
        copies.append(pltpu.async_copy(rel_hbm.at[p_idx.at[c]], p_rows.at[dst], sem))
        copies.append(pltpu.async_copy(ent_hbm.at[o_idx.at[c]], o_rows.at[dst], sem))
    for cp in copies:
        cp.wait()

    lane17 = lax.iota(jnp.int32, L) * 17

    def group_body(g, _):
        for l in range(L):
            j = g * L + l
            acc = (s_rows[j, pl.ds(0, L)]
                   * p_rows[j, pl.ds(0, L)]
                   * o_rows[j, pl.ds(0, L)])
            for k in range(1, D // L):
                sl = pl.ds(k * L, L)
                acc = acc + s_rows[j, sl] * p_rows[j, sl] * o_rows[j, sl]
            plsc.store_scatter(tmat, [lane17 + l], acc)
        res = tmat[pl.ds(0, L)]
        for d in range(1, L):
            res = res + tmat[pl.ds(d * 17, L)]
        out_v[pl.ds(g * L, L)] = res
        return ()

    lax.fori_loop(0, B_W // L, group_body, ())

    pltpu.sync_copy(out_v, out_hbm.at[wid])


def kernel(s, p, o, entity_emb, relation_emb):
    s2 = s.reshape(NW * NCHUNK, CB)
    p2 = p.reshape(NW * NCHUNK, CB)
    o2 = o.reshape(NW * NCHUNK, CB)
    mesh = plsc.VectorSubcoreMesh(core_axis_name="c", subcore_axis_name="s")
    run = pl.kernel(
        _kge_body,
        mesh=mesh,
        out_type=jax.ShapeDtypeStruct((NW, B_W), jnp.float32),
        compiler_params=pltpu.CompilerParams(
            needs_layout_passes=False, use_tc_tiling_on_sc=False),
        scratch_types=[
            pltpu.VMEM((NCHUNK, CB), jnp.int32),
            pltpu.VMEM((NCHUNK, CB), jnp.int32),
            pltpu.VMEM((NCHUNK, CB), jnp.int32),
            pltpu.VMEM((B_W, D), jnp.float32),
            pltpu.VMEM((B_W, D), jnp.float32),
            pltpu.VMEM((B_W, D), jnp.float32),
            pltpu.VMEM((B_W,), jnp.float32),
            pltpu.VMEM((L * 17,), jnp.float32),
            pltpu.SemaphoreType.DMA,
        ],
    )
    out = run(s2, p2, o2, entity_emb, relation_emb)
    return out.reshape(B, 1)
